# baseline (speedup 1.0000x reference)
import jax
import jax.numpy as jnp
from jax.experimental import pallas as pl
from jax.experimental.pallas import tpu as pltpu

N = 10000
F = 128
NOUT = 64
B = 200
NB = N // B


def _dot(a, b):
    return jax.lax.dot_general(a, b, (((1,), (0,)), ((), ())),
                               preferred_element_type=jnp.float32)


def _softmax_col(l):
    e = jnp.exp(l - jnp.max(l))
    return e / jnp.sum(e)


def _fused_kernel(adj_ref, x_in_ref, wn1_ref, ws1_ref, b1_ref,
                  wn2_ref, ws2_ref, b2_ref, g1_ref, g2_ref, g1b_ref, g2b_ref,
                  pw_ref, pb_ref,
                  zenc_ref, pred_ref,
                  x_ref, s_ref, r_ref, z_ref, acc_ref):
    p = pl.program_id(0)
    i = pl.program_id(1)

    @pl.when(p == 0)
    def _():
        xi = x_in_ref[...]
        s_ref[pl.ds(i * B, B), :] = _dot(xi, wn1_ref[...])
        x_ref[pl.ds(i * B, B), :] = _dot(xi, ws1_ref[...]) + b1_ref[...]

    @pl.when(p == 1)
    def _():
        x_ref[pl.ds(i * B, B), :] = jnp.maximum(
            _dot(adj_ref[...], s_ref[...]) + x_ref[pl.ds(i * B, B), :], 0.0)

    @pl.when((p == 2) & (i == 0))
    def _():
        x = x_ref[...]
        r_ref[...] = _softmax_col(_dot(x, g1_ref[...]) + g1b_ref[0, 0])
        z_ref[...] = _softmax_col(_dot(x, g2_ref[...]) + g2b_ref[0, 0])
        s_ref[...] = _dot(x * r_ref[...], wn2_ref[...])
        acc_ref[...] = jnp.zeros_like(acc_ref)

    @pl.when(p == 2)
    def _():
        x_blk = x_ref[pl.ds(i * B, B), :]
        r_blk = r_ref[pl.ds(i * B, B), :]
        z_blk = z_ref[pl.ds(i * B, B), :]
        h2 = _dot(x_blk * r_blk, ws2_ref[...]) + b2_ref[...]
        x2 = jnp.maximum(_dot(adj_ref[...], s_ref[...]) + h2, 0.0)
        zenc = (1.0 - z_blk) * x_blk + z_blk * x2
        zenc_ref[...] = zenc
        acc_ref[...] += jnp.sum(zenc, axis=0, keepdims=True)

    @pl.when((p == 2) & (i == NB - 1))
    def _():
        pred_ref[...] = _dot(acc_ref[...] * (1.0 / N), pw_ref[...]) + pb_ref[...]


def kernel(inputs, adj, Wn1, Ws1, b1, Wn2, Ws2, b2, g1w, g1b, g2w, g2b,
           e2pw, e2pb):
    f32 = jnp.float32
    full = lambda shape: pl.BlockSpec(shape, lambda p, i: (0,) * len(shape))
    adj_spec = pl.BlockSpec((B, N), lambda p, i: (jnp.minimum(p, 1) * i, 0))
    xin_spec = pl.BlockSpec(
        (B, F), lambda p, i: (jnp.where(p == 0, i, NB - 1), 0))
    zenc_spec = pl.BlockSpec((B, F), lambda p, i: ((p // 2) * i, 0))

    zenc, pred = pl.pallas_call(
        _fused_kernel,
        grid=(3, NB),
        in_specs=[adj_spec, xin_spec,
                  full((F, F)), full((F, F)), full((1, F)),
                  full((F, F)), full((F, F)), full((1, F)),
                  full((F, 1)), full((F, 1)), full((1, 1)), full((1, 1)),
                  full((F, NOUT)), full((1, NOUT))],
        out_specs=[zenc_spec, full((1, NOUT))],
        out_shape=[jax.ShapeDtypeStruct((N, F), f32),
                   jax.ShapeDtypeStruct((1, NOUT), f32)],
        scratch_shapes=[pltpu.VMEM((N, F), f32), pltpu.VMEM((N, F), f32),
                        pltpu.VMEM((N, 1), f32), pltpu.VMEM((N, 1), f32),
                        pltpu.VMEM((1, F), f32)],
        compiler_params=pltpu.CompilerParams(
            vmem_limit_bytes=100 * 1024 * 1024),
    )(adj, inputs, Wn1, Ws1, b1.reshape(1, F), Wn2, Ws2, b2.reshape(1, F),
      g1w, g2w, g1b.reshape(1, 1), g2b.reshape(1, 1), e2pw,
      e2pb.reshape(1, NOUT))

    return (zenc, pred)


# speedup vs baseline: 1.1108x; 1.1108x over previous
import jax
import jax.numpy as jnp
from jax.experimental import pallas as pl
from jax.experimental.pallas import tpu as pltpu

N = 10000
F = 128
NOUT = 64
B = 200
NB = N // B


def _dot(a, b):
    return jax.lax.dot_general(a, b, (((1,), (0,)), ((), ())),
                               preferred_element_type=jnp.float32)


def _softmax_col(l):
    e = jnp.exp(l - jnp.max(l))
    return e / jnp.sum(e)


def _fused_kernel(adj_ref, x_in_ref, wn1_ref, ws1_ref, b1_ref,
                  wn2_ref, ws2_ref, b2_ref, g1_ref, g2_ref, g1b_ref, g2b_ref,
                  pw_ref, pb_ref,
                  zenc_ref, pred_ref,
                  x_ref, s_ref, r_ref, z_ref, acc_ref):
    p = pl.program_id(0)
    i = pl.program_id(1)

    @pl.when((p == 0) & (i == 0))
    def _():
        s_ref[...] = _dot(x_in_ref[...], wn1_ref[...])

    @pl.when(p == 0)
    def _():
        rows = x_in_ref[pl.ds(i * B, B), :]
        h1 = _dot(rows, ws1_ref[...]) + b1_ref[...]
        x_ref[pl.ds(i * B, B), :] = jnp.maximum(
            _dot(adj_ref[...], s_ref[...]) + h1, 0.0)

    @pl.when((p == 1) & (i == 0))
    def _():
        x = x_ref[...]
        r_ref[...] = _softmax_col(_dot(x, g1_ref[...]) + g1b_ref[0, 0])
        z_ref[...] = _softmax_col(_dot(x, g2_ref[...]) + g2b_ref[0, 0])
        s_ref[...] = _dot(x * r_ref[...], wn2_ref[...])
        acc_ref[...] = jnp.zeros_like(acc_ref)

    @pl.when(p == 1)
    def _():
        x_blk = x_ref[pl.ds(i * B, B), :]
        r_blk = r_ref[pl.ds(i * B, B), :]
        z_blk = z_ref[pl.ds(i * B, B), :]
        h2 = _dot(x_blk * r_blk, ws2_ref[...]) + b2_ref[...]
        x2 = jnp.maximum(_dot(adj_ref[...], s_ref[...]) + h2, 0.0)
        zenc = (1.0 - z_blk) * x_blk + z_blk * x2
        zenc_ref[...] = zenc
        acc_ref[...] += jnp.sum(zenc, axis=0, keepdims=True)

    @pl.when((p == 1) & (i == NB - 1))
    def _():
        pred_ref[...] = _dot(acc_ref[...] * (1.0 / N), pw_ref[...]) + pb_ref[...]


def kernel(inputs, adj, Wn1, Ws1, b1, Wn2, Ws2, b2, g1w, g1b, g2w, g2b,
           e2pw, e2pb):
    f32 = jnp.float32
    full = lambda shape: pl.BlockSpec(shape, lambda p, i: (0,) * len(shape))

    zenc, pred = pl.pallas_call(
        _fused_kernel,
        grid=(2, NB),
        in_specs=[pl.BlockSpec((B, N), lambda p, i: (i, 0)),
                  full((N, F)), full((F, F)), full((F, F)), full((1, F)),
                  full((F, F)), full((F, F)), full((1, F)),
                  full((F, 1)), full((F, 1)), full((1, 1)), full((1, 1)),
                  full((F, NOUT)), full((1, NOUT))],
        out_specs=[pl.BlockSpec((B, F), lambda p, i: (p * i, 0)),
                   full((1, NOUT))],
        out_shape=[jax.ShapeDtypeStruct((N, F), f32),
                   jax.ShapeDtypeStruct((1, NOUT), f32)],
        scratch_shapes=[pltpu.VMEM((N, F), f32), pltpu.VMEM((N, F), f32),
                        pltpu.VMEM((N, 1), f32), pltpu.VMEM((N, 1), f32),
                        pltpu.VMEM((1, F), f32)],
        compiler_params=pltpu.CompilerParams(
            vmem_limit_bytes=100 * 1024 * 1024),
    )(adj, inputs, Wn1, Ws1, b1.reshape(1, F), Wn2, Ws2, b2.reshape(1, F),
      g1w, g2w, g1b.reshape(1, 1), g2b.reshape(1, 1), e2pw,
      e2pb.reshape(1, NOUT))

    return (zenc, pred)
